# baseline (speedup 1.0000x reference)
import jax
import jax.numpy as jnp
from jax import lax
from jax.experimental import pallas as pl
from jax.experimental.pallas import tpu as pltpu

_BN_EPS = 1e-5
_VMEM_LIMIT = 48 * 1024 * 1024
_TR = 112


def _make_pool_stats_kernel(c):
    def _body(x_ref, pooled_ref, stats_ref):
        x = x_ref[...].astype(jnp.float32)
        xw = x[..., :c] + x[..., c:]
        pooled = (xw[:, 0] + xw[:, 1]) * 0.25
        t, wo, cc = pooled.shape
        pooled_ref[...] = pooled.reshape(t * wo, cc).astype(pooled_ref.dtype)

        s = jnp.sum(x, axis=(0, 1, 2))[None, :]
        ss = jnp.sum(x * x, axis=(0, 1, 2))[None, :]
        stats_ref[0] = jnp.concatenate([s, ss], axis=0)

    return _body


def _make_matmul_kernel(c, cnt):
    def _body(p_ref, w_ref, stats_ref, g_ref, b_ref, o_ref):
        sums2 = jnp.sum(stats_ref[...], axis=0)
        sums = sums2[:, :c] + sums2[:, c:]
        mean = sums[0:1] / cnt
        var = jnp.maximum(sums[1:2] / cnt - mean * mean, 0.0)
        scale = g_ref[...] * lax.rsqrt(var + _BN_EPS)
        off = b_ref[...] - mean * scale
        for j in range(p_ref.shape[0]):
            q = (p_ref[j].astype(jnp.float32) * scale + off).astype(w_ref.dtype)
            o_ref[j] = lax.dot_general(
                w_ref[...], q, (((1,), (1,)), ((), ())),
                preferred_element_type=jnp.float32).astype(o_ref.dtype)

    return _body


def kernel(x_nchw, w_oc, gamma, beta):
    N, C, H, W = x_nchw.shape
    Cout = w_oc.shape[0]
    Ho, Wo = H // 2, W // 2
    P = Ho * Wo

    x_nhwc = jnp.transpose(x_nchw, (0, 2, 3, 1)).astype(jnp.float32)
    x4 = x_nhwc.reshape(N * Ho, 2, Wo, 2 * C)

    rows = N * Ho
    tr = _TR if rows % _TR == 0 else 1
    grid = rows // tr

    pooled, stats = pl.pallas_call(
        _make_pool_stats_kernel(C),
        out_shape=(
            jax.ShapeDtypeStruct((rows * Wo, C), jnp.bfloat16),
            jax.ShapeDtypeStruct((grid, 2, 2 * C), jnp.float32),
        ),
        grid=(grid,),
        in_specs=[pl.BlockSpec((tr, 2, Wo, 2 * C), lambda i: (i, 0, 0, 0))],
        out_specs=(
            pl.BlockSpec((tr * Wo, C), lambda i: (i, 0)),
            pl.BlockSpec((1, 2, 2 * C), lambda i: (i, 0, 0)),
        ),
        compiler_params=pltpu.CompilerParams(
            dimension_semantics=("parallel",),
            vmem_limit_bytes=_VMEM_LIMIT),
    )(x4)

    w_bf = w_oc.astype(jnp.bfloat16)
    out = pl.pallas_call(
        _make_matmul_kernel(C, float(N * H * W)),
        out_shape=jax.ShapeDtypeStruct((N, Cout, P), jnp.float32),
        grid=(N // 8,),
        in_specs=[
            pl.BlockSpec((8, P, C), lambda i: (i, 0, 0)),
            pl.BlockSpec((Cout, C), lambda i: (0, 0)),
            pl.BlockSpec((grid, 2, 2 * C), lambda i: (0, 0, 0)),
            pl.BlockSpec((1, C), lambda i: (0, 0)),
            pl.BlockSpec((1, C), lambda i: (0, 0)),
        ],
        out_specs=pl.BlockSpec((8, Cout, P), lambda i: (i, 0, 0)),
        compiler_params=pltpu.CompilerParams(
            dimension_semantics=("parallel",),
            vmem_limit_bytes=_VMEM_LIMIT),
    )(pooled.reshape(N, P, C), w_bf, stats,
      gamma.astype(jnp.float32).reshape(1, C),
      beta.astype(jnp.float32).reshape(1, C))

    return out.reshape(N, Cout, Ho, Wo).astype(x_nchw.dtype)


# speedup vs baseline: 1.0303x; 1.0303x over previous
import jax
import jax.numpy as jnp
from jax import lax
from jax.experimental import pallas as pl
from jax.experimental.pallas import tpu as pltpu

_BN_EPS = 1e-5
_VMEM_LIMIT = 48 * 1024 * 1024
_TR = 64


def _make_pool_stats_kernel(c):
    def _body(x_ref, pooled_ref, stats_ref):
        x = x_ref[...].astype(jnp.float32)
        xw = x[..., :c] + x[..., c:]
        pooled = (xw[:, 0] + xw[:, 1]) * 0.25
        t, wo, cc = pooled.shape
        pooled_ref[...] = pooled.reshape(t * wo, cc).astype(pooled_ref.dtype)

        s = jnp.sum(x, axis=(0, 1, 2))[None, :]
        ss = jnp.sum(x * x, axis=(0, 1, 2))[None, :]
        stats_ref[0] = jnp.concatenate([s, ss], axis=0)

    return _body


def _make_matmul_kernel(c, cnt):
    def _body(p_ref, w_ref, stats_ref, g_ref, b_ref, o_ref):
        sums2 = jnp.sum(stats_ref[...], axis=0)
        sums = sums2[:, :c] + sums2[:, c:]
        mean = sums[0:1] / cnt
        var = jnp.maximum(sums[1:2] / cnt - mean * mean, 0.0)
        scale = g_ref[...] * lax.rsqrt(var + _BN_EPS)
        off = b_ref[...] - mean * scale
        for j in range(p_ref.shape[0]):
            q = (p_ref[j].astype(jnp.float32) * scale + off).astype(w_ref.dtype)
            o_ref[j] = lax.dot_general(
                w_ref[...], q, (((1,), (1,)), ((), ())),
                preferred_element_type=jnp.float32).astype(o_ref.dtype)

    return _body


def kernel(x_nchw, w_oc, gamma, beta):
    N, C, H, W = x_nchw.shape
    Cout = w_oc.shape[0]
    Ho, Wo = H // 2, W // 2
    P = Ho * Wo

    x_nhwc = jnp.transpose(x_nchw, (0, 2, 3, 1)).astype(jnp.float32)
    x4 = x_nhwc.reshape(N * Ho, 2, Wo, 2 * C)

    rows = N * Ho
    tr = _TR if rows % _TR == 0 else 1
    grid = rows // tr

    pooled, stats = pl.pallas_call(
        _make_pool_stats_kernel(C),
        out_shape=(
            jax.ShapeDtypeStruct((rows * Wo, C), jnp.bfloat16),
            jax.ShapeDtypeStruct((grid, 2, 2 * C), jnp.float32),
        ),
        grid=(grid,),
        in_specs=[pl.BlockSpec((tr, 2, Wo, 2 * C), lambda i: (i, 0, 0, 0))],
        out_specs=(
            pl.BlockSpec((tr * Wo, C), lambda i: (i, 0)),
            pl.BlockSpec((1, 2, 2 * C), lambda i: (i, 0, 0)),
        ),
        compiler_params=pltpu.CompilerParams(
            dimension_semantics=("parallel",),
            vmem_limit_bytes=_VMEM_LIMIT),
    )(x4)

    w_bf = w_oc.astype(jnp.bfloat16)
    out = pl.pallas_call(
        _make_matmul_kernel(C, float(N * H * W)),
        out_shape=jax.ShapeDtypeStruct((N, Cout, P), jnp.bfloat16),
        grid=(N // 8,),
        in_specs=[
            pl.BlockSpec((8, P, C), lambda i: (i, 0, 0)),
            pl.BlockSpec((Cout, C), lambda i: (0, 0)),
            pl.BlockSpec((grid, 2, 2 * C), lambda i: (0, 0, 0)),
            pl.BlockSpec((1, C), lambda i: (0, 0)),
            pl.BlockSpec((1, C), lambda i: (0, 0)),
        ],
        out_specs=pl.BlockSpec((8, Cout, P), lambda i: (i, 0, 0)),
        compiler_params=pltpu.CompilerParams(
            dimension_semantics=("parallel",),
            vmem_limit_bytes=_VMEM_LIMIT),
    )(pooled.reshape(N, P, C), w_bf, stats,
      gamma.astype(jnp.float32).reshape(1, C),
      beta.astype(jnp.float32).reshape(1, C))

    return out.reshape(N, Cout, Ho, Wo).astype(x_nchw.dtype)
